# baseline (speedup 1.0000x reference)
import dataclasses

import jax
import jax.numpy as jnp
from jax import lax
from jax.experimental import pallas as pl
from jax.experimental.pallas import tpu as pltpu
from jax.experimental.pallas import tpu_sc as plsc

N = 10000
K = 16
L = 16
NC = 2
NS = 16
NW = NC * NS
NPW = 320
NB = NPW // L


def _arap_tec(pc_hbm, idx_hbm, dist_hbm, out_hbm,
              pcf_v, bidx_v, bdist_v, acc_v):
    cid = lax.axis_index("c")
    sid = lax.axis_index("s")
    wid = sid * NC + cid
    start = jnp.minimum(wid * NPW, N - NPW)
    b_lo = jnp.where(wid == NW - 1, NB - (N - (NW - 1) * NPW) // L, 0)

    pltpu.sync_copy(pc_hbm, pcf_v)
    pltpu.sync_copy(idx_hbm.at[pl.ds(start * K, NPW * K)], bidx_v)
    pltpu.sync_copy(dist_hbm.at[pl.ds(start * K, NPW * K)], bdist_v)

    iota = lax.iota(jnp.int32, L)
    iotaK = iota * K
    iota3 = iota * 3
    zero = jnp.zeros((L,), jnp.float32)

    def outer(b, accs):
        acc0, acc1 = accs
        p0 = b * L
        base = iotaK + p0 * K
        cbase = iota3 + (start + p0) * 3
        cx = plsc.load_gather(pcf_v, [cbase])
        cy = plsc.load_gather(pcf_v, [cbase + 1])
        cz = plsc.load_gather(pcf_v, [cbase + 2])
        for k in range(K):
            ev = base + k
            idxv = plsc.load_gather(bidx_v, [ev])
            distv = plsc.load_gather(bdist_v, [ev])
            idx3 = idxv * 3
            gx = plsc.load_gather(pcf_v, [idx3])
            gy = plsc.load_gather(pcf_v, [idx3 + 1])
            gz = plsc.load_gather(pcf_v, [idx3 + 2])
            dx = cx - gx
            dy = cy - gy
            dz = cz - gz
            d2 = dx * dx + dy * dy + dz * dz
            term = jnp.abs(d2 - distv)
            if k % 2 == 0:
                acc0 = acc0 + term
            else:
                acc1 = acc1 + term
        return acc0, acc1

    acc0, acc1 = plsc.parallel_loop(b_lo, NB, carry=(zero, zero))(outer)
    acc_v[...] = acc0 + acc1
    pltpu.sync_copy(acc_v, out_hbm.at[wid])


@jax.jit
def _arap_sc(pcf, idxf, distf):
    cp = pltpu.CompilerParams(use_tc_tiling_on_sc=False)
    if "needs_layout_passes" in pltpu.CompilerParams.__dataclass_fields__:
        cp = dataclasses.replace(cp, needs_layout_passes=False)
    run = pl.kernel(
        _arap_tec,
        out_type=jax.ShapeDtypeStruct((NW, L), jnp.float32),
        compiler_params=cp,
        mesh=plsc.VectorSubcoreMesh(core_axis_name="c", subcore_axis_name="s"),
        scratch_types=[
            pltpu.VMEM((3 * N,), jnp.float32),
            pltpu.VMEM((K * NPW,), jnp.int32),
            pltpu.VMEM((K * NPW,), jnp.float32),
            pltpu.VMEM((L,), jnp.float32),
        ],
    )
    return run(pcf, idxf, distf)


def kernel(pc_transformed, nn_distances, nn_indices):
    if nn_indices.dtype != jnp.int32:
        nn_indices = nn_indices.astype(jnp.int32)
    partials = _arap_sc(pc_transformed.reshape(-1),
                        nn_indices.reshape(-1),
                        nn_distances.reshape(-1))
    return jnp.sum(partials) / (N * K)


# speedup vs baseline: 1.4815x; 1.4815x over previous
import dataclasses

import jax
import jax.numpy as jnp
from jax import lax
from jax.experimental import pallas as pl
from jax.experimental.pallas import tpu as pltpu
from jax.experimental.pallas import tpu_sc as plsc

N = 10000
K = 16
L = 16
NC = 2
NS = 16
NW = NC * NS
NPW = 320
NB = NPW // L


def _arap_tec(pc_hbm, idx_hbm, dist_hbm, out_hbm,
              pcx_v, pcy_v, pcz_v, bidx_v, bdist_v, acc_v, sem):
    cid = lax.axis_index("c")
    sid = lax.axis_index("s")
    wid = sid * NC + cid
    start = jnp.minimum(wid * NPW, N - NPW)
    b_lo = jnp.where(wid == NW - 1, NB - (N - (NW - 1) * NPW) // L, 0)

    copies = [
        pltpu.async_copy(pc_hbm.at[0], pcx_v, sem),
        pltpu.async_copy(pc_hbm.at[1], pcy_v, sem),
        pltpu.async_copy(pc_hbm.at[2], pcz_v, sem),
        pltpu.async_copy(idx_hbm.at[:, pl.ds(start, NPW)], bidx_v, sem),
        pltpu.async_copy(dist_hbm.at[:, pl.ds(start, NPW)], bdist_v, sem),
    ]
    for c in copies:
        c.wait()

    def outer(b, accs):
        acc0, acc1 = accs
        p0 = b * L
        cx = pcx_v[pl.ds(start + p0, L)]
        cy = pcy_v[pl.ds(start + p0, L)]
        cz = pcz_v[pl.ds(start + p0, L)]
        for k in range(K):
            idxv = bidx_v[k, pl.ds(p0, L)]
            gx = plsc.load_gather(pcx_v, [idxv])
            gy = plsc.load_gather(pcy_v, [idxv])
            gz = plsc.load_gather(pcz_v, [idxv])
            dx = cx - gx
            dy = cy - gy
            dz = cz - gz
            d2 = dx * dx + dy * dy + dz * dz
            term = jnp.abs(d2 - bdist_v[k, pl.ds(p0, L)])
            if k % 2 == 0:
                acc0 = acc0 + term
            else:
                acc1 = acc1 + term
        return acc0, acc1

    zero = jnp.zeros((L,), jnp.float32)
    acc0, acc1 = plsc.parallel_loop(b_lo, NB, carry=(zero, zero))(outer)
    acc_v[...] = acc0 + acc1
    pltpu.sync_copy(acc_v, out_hbm.at[wid])


@jax.jit
def _arap_sc(pcT, idxT, distT):
    cp = pltpu.CompilerParams(use_tc_tiling_on_sc=False)
    if "needs_layout_passes" in pltpu.CompilerParams.__dataclass_fields__:
        cp = dataclasses.replace(cp, needs_layout_passes=False)
    run = pl.kernel(
        _arap_tec,
        out_type=jax.ShapeDtypeStruct((NW, L), jnp.float32),
        compiler_params=cp,
        mesh=plsc.VectorSubcoreMesh(core_axis_name="c", subcore_axis_name="s"),
        scratch_types=[
            pltpu.VMEM((N,), jnp.float32),
            pltpu.VMEM((N,), jnp.float32),
            pltpu.VMEM((N,), jnp.float32),
            pltpu.VMEM((K, NPW), jnp.int32),
            pltpu.VMEM((K, NPW), jnp.float32),
            pltpu.VMEM((L,), jnp.float32),
            pltpu.SemaphoreType.DMA,
        ],
    )
    return run(pcT, idxT, distT)


def kernel(pc_transformed, nn_distances, nn_indices):
    if nn_indices.dtype != jnp.int32:
        nn_indices = nn_indices.astype(jnp.int32)
    partials = _arap_sc(pc_transformed.T, nn_indices.T, nn_distances.T)
    return jnp.sum(partials) / (N * K)
